# baseline (speedup 1.0000x reference)
import numpy as np
import jax
import jax.numpy as jnp
from jax.experimental import pallas as pl
from jax.experimental.pallas import tpu as pltpu

_EPS = 1e-6
_P = 6.5
_BINS = 64


def _body(a_ref, x_ref, o_ref):
    z = jnp.maximum(x_ref[...], _EPS)
    zp = jnp.exp2(_P * jnp.log2(z)).astype(jnp.bfloat16)
    a = a_ref[...]
    for b in range(x_ref.shape[0]):
        m = jax.lax.dot_general(a, zp[b], (((1,), (0,)), ((), ())),
                                preferred_element_type=jnp.float32)
        o_ref[b, :, :] = jnp.exp2(jnp.log2(m) * (1.0 / _P))


def _pool_matrix(hw):
    k = hw // _BINS
    a = np.repeat(np.eye(_BINS, dtype=np.float32), k, axis=0).T / float(k)
    return jnp.asarray(a, dtype=jnp.bfloat16)


def kernel(x):
    n, c, h, w = x.shape
    hw = h * w
    assert hw % _BINS == 0

    x3 = x.transpose(0, 2, 3, 1).reshape(n, hw, c)

    bn = 32
    while n % bn:
        bn //= 2
    grid = (n // bn,)

    out3 = pl.pallas_call(
        _body,
        out_shape=jax.ShapeDtypeStruct((n, _BINS, c), x.dtype),
        grid=grid,
        in_specs=[
            pl.BlockSpec((_BINS, hw), lambda i: (0, 0)),
            pl.BlockSpec((bn, hw, c), lambda i: (i, 0, 0)),
        ],
        out_specs=pl.BlockSpec((bn, _BINS, c), lambda i: (i, 0, 0)),
        compiler_params=pltpu.CompilerParams(
            dimension_semantics=("parallel",),
            vmem_limit_bytes=48 * 1024 * 1024,
        ),
    )(_pool_matrix(hw), x3)

    return out3.transpose(0, 2, 1)


# speedup vs baseline: 1.0748x; 1.0748x over previous
import numpy as np
import jax
import jax.numpy as jnp
from jax.experimental import pallas as pl
from jax.experimental.pallas import tpu as pltpu

_EPS = 1e-6
_P = 6.5
_BINS = 64


def _body(a_ref, x_ref, o_ref):
    z = jnp.maximum(x_ref[...], _EPS)
    zp = jnp.exp2(_P * jnp.log2(z)).astype(jnp.bfloat16)
    a = a_ref[...]
    for b in range(x_ref.shape[0]):
        m = jax.lax.dot_general(a, zp[b], (((1,), (0,)), ((), ())),
                                preferred_element_type=jnp.float32)
        o_ref[b, :, :] = jnp.exp2(jnp.log2(m) * (1.0 / _P))


def _pool_matrix(hw):
    k = hw // _BINS
    a = np.repeat(np.eye(_BINS, dtype=np.float32), k, axis=0).T / float(k)
    return jnp.asarray(a, dtype=jnp.bfloat16)


def kernel(x):
    n, c, h, w = x.shape
    hw = h * w
    assert hw % _BINS == 0

    x3 = x.transpose(0, 2, 3, 1).reshape(n, hw, c)

    bn = 16
    while n % bn:
        bn //= 2
    grid = (n // bn,)

    out3 = pl.pallas_call(
        _body,
        out_shape=jax.ShapeDtypeStruct((n, _BINS, c), x.dtype),
        grid=grid,
        in_specs=[
            pl.BlockSpec((_BINS, hw), lambda i: (0, 0)),
            pl.BlockSpec((bn, hw, c), lambda i: (i, 0, 0)),
        ],
        out_specs=pl.BlockSpec((bn, _BINS, c), lambda i: (i, 0, 0)),
        compiler_params=pltpu.CompilerParams(
            dimension_semantics=("parallel",),
            vmem_limit_bytes=48 * 1024 * 1024,
        ),
    )(_pool_matrix(hw), x3)

    return out3.transpose(0, 2, 1)
